# baseline (speedup 1.0000x reference)
import functools

import jax
import jax.numpy as jnp
from jax import lax
from jax.experimental import pallas as pl
from jax.experimental.pallas import tpu as pltpu
from jax.experimental.pallas import tpu_sc as plsc

_ROW = 128
_FRAME_ROWS = 512
_HALF = _FRAME_ROWS // 2


def _sc_copy(table, base):
    mesh = plsc.ScalarSubcoreMesh(axis_name="c", num_cores=2)

    @functools.partial(
        pl.kernel,
        mesh=mesh,
        out_type=jax.ShapeDtypeStruct((_FRAME_ROWS, _ROW), jnp.float32),
        scratch_types=[pltpu.SMEM((16,), jnp.int32)],
    )
    def k(table_hbm, base_hbm, out_hbm, base_s):
        cid = lax.axis_index("c")
        pltpu.sync_copy(base_hbm, base_s)
        j = pl.multiple_of(base_s[0] + cid * _HALF, 8)
        pltpu.sync_copy(
            table_hbm.at[pl.ds(j, _HALF)],
            out_hbm.at[pl.ds(cid * _HALF, _HALF)],
        )

    return k(table, base)


def kernel(noise, idx):
    length = noise.shape[0]
    table = noise.reshape(length * _FRAME_ROWS, _ROW)
    base = jnp.full((16,), 0, jnp.int32).at[0].set(
        (jnp.asarray(idx, jnp.int32) % length) * _FRAME_ROWS
    )
    out = _sc_copy(table, base)
    return out.reshape(1, *noise.shape[1:])


# speedup vs baseline: 2.3330x; 2.3330x over previous
import functools

import jax
import jax.numpy as jnp
from jax import lax
from jax.experimental import pallas as pl
from jax.experimental.pallas import tpu as pltpu
from jax.experimental.pallas import tpu_sc as plsc

_HALF = 128


def _sc_copy(noise, base):
    mesh = plsc.ScalarSubcoreMesh(axis_name="c", num_cores=2)

    @functools.partial(
        pl.kernel,
        mesh=mesh,
        out_type=jax.ShapeDtypeStruct((1, 1, 256, 256), jnp.float32),
        scratch_types=[pltpu.SMEM((16,), jnp.int32)],
    )
    def k(noise_hbm, base_hbm, out_hbm, base_s):
        cid = lax.axis_index("c")
        pltpu.sync_copy(base_hbm, base_s)
        j = base_s[0]
        r = pl.multiple_of(cid * _HALF, 8)
        pltpu.sync_copy(
            noise_hbm.at[j, 0, pl.ds(r, _HALF)],
            out_hbm.at[0, 0, pl.ds(r, _HALF)],
        )

    return k(noise, base)


def kernel(noise, idx):
    length = noise.shape[0]
    base = jnp.full((16,), jnp.asarray(idx, jnp.int32) % length, jnp.int32)
    return _sc_copy(noise, base)


# speedup vs baseline: 2.4685x; 1.0581x over previous
import functools

import jax
import jax.numpy as jnp
from jax import lax
from jax.experimental import pallas as pl
from jax.experimental.pallas import tpu as pltpu
from jax.experimental.pallas import tpu_sc as plsc

_HALF = 128


def _sc_copy(noise, base):
    mesh = plsc.ScalarSubcoreMesh(axis_name="c", num_cores=1)

    @functools.partial(
        pl.kernel,
        mesh=mesh,
        out_type=jax.ShapeDtypeStruct((1, 1, 256, 256), jnp.float32),
        scratch_types=[pltpu.SMEM((16,), jnp.int32)],
    )
    def k(noise_hbm, base_hbm, out_hbm, base_s):
        pltpu.sync_copy(base_hbm, base_s)
        j = base_s[0]
        pltpu.sync_copy(noise_hbm.at[j, 0], out_hbm.at[0, 0])

    return k(noise, base)


def kernel(noise, idx):
    length = noise.shape[0]
    base = jnp.full((16,), jnp.asarray(idx, jnp.int32) % length, jnp.int32)
    return _sc_copy(noise, base)


# speedup vs baseline: 2.9806x; 1.2075x over previous
import functools

import jax
import jax.numpy as jnp
from jax import lax
from jax.experimental import pallas as pl
from jax.experimental.pallas import tpu as pltpu
from jax.experimental.pallas import tpu_sc as plsc

_ROW = 256
_FRAME_ROWS = 256
_NW = 32
_RPW = _FRAME_ROWS // _NW


def _sc_gather(table, row_idx):
    mesh = plsc.VectorSubcoreMesh(core_axis_name="c", subcore_axis_name="s")

    @functools.partial(
        pl.kernel,
        mesh=mesh,
        out_type=jax.ShapeDtypeStruct((_FRAME_ROWS, _ROW), jnp.float32),
        scratch_types=[
            pltpu.VMEM((_RPW,), jnp.int32),
            pltpu.VMEM((_RPW, _ROW), jnp.float32),
            pltpu.SemaphoreType.DMA,
        ],
    )
    def k(table_hbm, idx_hbm, out_hbm, idx_v, rows_v, sem):
        wid = lax.axis_index("s") * 2 + lax.axis_index("c")
        pltpu.sync_copy(idx_hbm.at[wid], idx_v)
        pltpu.async_copy(table_hbm.at[idx_v], rows_v, sem).wait()
        pltpu.sync_copy(rows_v, out_hbm.at[pl.ds(wid * _RPW, _RPW)])

    return k(table, row_idx)


def kernel(noise, idx):
    length = noise.shape[0]
    table = noise.reshape(length * _FRAME_ROWS, _ROW)
    base = (jnp.asarray(idx, jnp.int32) % length) * _FRAME_ROWS
    row_idx = (base + jnp.arange(_FRAME_ROWS, dtype=jnp.int32)).reshape(
        _NW, _RPW
    )
    out = _sc_gather(table, row_idx)
    return out.reshape(1, *noise.shape[1:])


# speedup vs baseline: 3.1817x; 1.0675x over previous
import functools

import jax
import jax.numpy as jnp
from jax import lax
from jax.experimental import pallas as pl
from jax.experimental.pallas import tpu as pltpu
from jax.experimental.pallas import tpu_sc as plsc

_ROW = 256
_FRAME_ROWS = 256
_NW = 16
_RPW = _FRAME_ROWS // _NW


def _sc_gather(table, row_idx):
    mesh = plsc.VectorSubcoreMesh(
        core_axis_name="c", subcore_axis_name="s", num_cores=1
    )

    @functools.partial(
        pl.kernel,
        mesh=mesh,
        out_type=jax.ShapeDtypeStruct((_FRAME_ROWS, _ROW), jnp.float32),
        scratch_types=[
            pltpu.VMEM((_RPW,), jnp.int32),
            pltpu.VMEM((_RPW, _ROW), jnp.float32),
            pltpu.SemaphoreType.DMA,
        ],
    )
    def k(table_hbm, idx_hbm, out_hbm, idx_v, rows_v, sem):
        wid = lax.axis_index("s")
        pltpu.sync_copy(idx_hbm.at[wid], idx_v)
        pltpu.async_copy(table_hbm.at[idx_v], rows_v, sem).wait()
        pltpu.sync_copy(rows_v, out_hbm.at[pl.ds(wid * _RPW, _RPW)])

    return k(table, row_idx)


def kernel(noise, idx):
    length = noise.shape[0]
    table = noise.reshape(length * _FRAME_ROWS, _ROW)
    base = (jnp.asarray(idx, jnp.int32) % length) * _FRAME_ROWS
    row_idx = (base + jnp.arange(_FRAME_ROWS, dtype=jnp.int32)).reshape(
        _NW, _RPW
    )
    out = _sc_gather(table, row_idx)
    return out.reshape(1, *noise.shape[1:])
